# Initial kernel scaffold; baseline (speedup 1.0000x reference)
import jax
import jax.numpy as jnp
from jax.experimental import pallas as pl


def kernel(x, edge_index, edge_label_index, W1, b1, W2, b2):
    raise NotImplementedError("write your pallas kernel here")


# speedup vs baseline: 4.5526x; 4.5526x over previous
import functools

import jax
import jax.numpy as jnp
from jax import lax
from jax.experimental import pallas as pl
from jax.experimental.pallas import tpu as pltpu
from jax.experimental.pallas import tpu_sc as plsc

N = 10000
E = 160000
EROWS = 1280
EP = EROWS * 128
NACC = 10112
NB = 10
NBS = 1000
NC = 2
NS = 16

_mesh = plsc.VectorSubcoreMesh(
    core_axis_name="c", subcore_axis_name="s", num_cores=NC, num_subcores=NS)


def _sc_deg():
    RT = (EP // 64) // (NC * NS)
    ZR = NACC // NS

    @functools.partial(
        pl.kernel,
        mesh=_mesh,
        compiler_params=pltpu.CompilerParams(needs_layout_passes=False),
        out_type=jax.ShapeDtypeStruct((NC, NACC, 16), jnp.float32),
        scratch_types=[
            pltpu.VMEM_SHARED((NACC, 16), jnp.float32),
            pltpu.VMEM((ZR, 16), jnp.float32),
            pltpu.VMEM((64, 16), jnp.float32),
            pltpu.VMEM((RT, 64), jnp.int32),
            pltpu.SemaphoreType.DMA,
        ],
    )
    def deg_kernel(dst_hbm, out_hbm, acc, zbuf, vals, idx, sem):
        c = lax.axis_index("c")
        s = lax.axis_index("s")
        w = s * NC + c

        def init_body(i, _):
            zbuf[i, :] = jnp.zeros((16,), jnp.float32)
            return 0
        lax.fori_loop(0, ZR, init_body, 0)

        def ones_body(i, _):
            vals[i, :] = jnp.ones((16,), jnp.float32)
            return 0
        lax.fori_loop(0, 64, ones_body, 0)

        pltpu.sync_copy(zbuf, acc.at[pl.ds(s * ZR, ZR)])
        plsc.subcore_barrier()

        pltpu.sync_copy(dst_hbm.at[pl.ds(w * RT, RT)], idx)

        def edge_body(j, _):
            pltpu.async_copy(vals, acc.at[idx.at[j]], sem, add=True).wait()
            return 0
        lax.fori_loop(0, RT, edge_body, 0)

        plsc.subcore_barrier()
        pltpu.sync_copy(acc.at[pl.ds(s * ZR, ZR)],
                        out_hbm.at[c].at[pl.ds(s * ZR, ZR)])

    return deg_kernel


def _sc_agg(C):
    CPC = C // NC
    RT = (EP // 64) // NS
    K = 32
    NSB = RT // K
    D = 4
    ZR = 64
    ZT = NACC // NS
    WB = 624

    @functools.partial(
        pl.kernel,
        mesh=_mesh,
        compiler_params=pltpu.CompilerParams(needs_layout_passes=False),
        out_type=jax.ShapeDtypeStruct((C * N, 128), jnp.float32),
        scratch_types=[
            pltpu.VMEM_SHARED((NACC, 128), jnp.float32),
            pltpu.VMEM((ZR, 128), jnp.float32),
            pltpu.VMEM((K, 64), jnp.int32),
            pltpu.VMEM((K, 64), jnp.int32),
            [pltpu.VMEM((64, 128), jnp.float32)] * D,
            [pltpu.SemaphoreType.DMA] * D,
            [pltpu.SemaphoreType.DMA] * D,
        ],
    )
    def agg_kernel(hs_hbm, src_hbm, dst_hbm, out_hbm,
                   acc, zbuf, idxs, idxd, bufs, semg, sems):
        c = lax.axis_index("c")
        s = lax.axis_index("s")

        def zb_body(i, _):
            for k8 in range(8):
                zbuf[i, pl.ds(k8 * 16, 16)] = jnp.zeros((16,), jnp.float32)
            return 0
        lax.fori_loop(0, ZR, zb_body, 0)

        for chunk in range(CPC):
            fc = c * CPC + chunk
            for z in range(9):
                pltpu.sync_copy(zbuf, acc.at[pl.ds(s * ZT + z * ZR, ZR)])
            pltpu.sync_copy(zbuf.at[pl.ds(0, 56)],
                            acc.at[pl.ds(s * ZT + 9 * ZR, 56)])
            plsc.subcore_barrier()

            off = fc * N

            def sb_body(t, _):
                r0 = s * RT + t * K
                pltpu.sync_copy(src_hbm.at[pl.ds(r0, K)], idxs)
                pltpu.sync_copy(dst_hbm.at[pl.ds(r0, K)], idxd)
                for j in range(K):
                    for k4 in range(4):
                        idxs[j, pl.ds(k4 * 16, 16)] = (
                            idxs[j, pl.ds(k4 * 16, 16)] + off)
                g = [None] * K
                sd = [None] * K
                for j in range(D):
                    g[j] = pltpu.async_copy(
                        hs_hbm.at[idxs.at[j]], bufs[j], semg[j])
                for j in range(K):
                    b = j % D
                    g[j].wait()
                    sd[j] = pltpu.async_copy(
                        bufs[b], acc.at[idxd.at[j]], sems[b], add=True)
                    k = j - (D - 1)
                    if k >= 0 and k + D < K:
                        sd[k].wait()
                        g[k + D] = pltpu.async_copy(
                            hs_hbm.at[idxs.at[k + D]], bufs[k % D], semg[k % D])
                for j in range(K - D, K):
                    sd[j].wait()
                return 0
            lax.fori_loop(0, NSB, sb_body, 0)

            plsc.subcore_barrier()
            pltpu.sync_copy(acc.at[pl.ds(s * WB, WB)],
                            out_hbm.at[pl.ds(off + s * WB, WB)])

            @pl.when(s == NS - 1)
            def _():
                pltpu.sync_copy(acc.at[pl.ds(NS * WB, N - NS * WB)],
                                out_hbm.at[pl.ds(off + NS * WB, N - NS * WB)])
            if chunk + 1 < CPC:
                plsc.subcore_barrier()

    return agg_kernel


def _sc_decode():
    G = 16
    D = 4
    NBT = EP // (NC * NS * G)
    IR = NBT // 8
    EB = NBT * G

    @functools.partial(
        pl.kernel,
        mesh=_mesh,
        compiler_params=pltpu.CompilerParams(needs_layout_passes=False),
        out_type=jax.ShapeDtypeStruct((EP,), jnp.float32),
        scratch_types=[
            pltpu.VMEM((IR, 128), jnp.int32),
            pltpu.VMEM((IR, 128), jnp.int32),
            [pltpu.VMEM((G, 256), jnp.float32)] * D,
            [pltpu.VMEM((G, 256), jnp.float32)] * D,
            pltpu.VMEM((EB,), jnp.float32),
            [pltpu.SemaphoreType.DMA] * D,
            [pltpu.SemaphoreType.DMA] * D,
        ],
    )
    def dec_kernel(z2_hbm, els_hbm, eld_hbm, out_hbm,
                   idxs, idxd, zsb, zdb, outbuf, sems, semd):
        c = lax.axis_index("c")
        s = lax.axis_index("s")
        w = s * NC + c
        pltpu.sync_copy(els_hbm.at[pl.ds(w * IR, IR)], idxs)
        pltpu.sync_copy(eld_hbm.at[pl.ds(w * IR, IR)], idxd)

        iota = lax.iota(jnp.int32, 16)

        def ib(ref, b):
            return ref.at[b // 8, pl.ds((b % 8) * 16, 16)]

        for p in range(D):
            pltpu.async_copy(z2_hbm.at[ib(idxs, p)], zsb[p], sems[p])
            pltpu.async_copy(z2_hbm.at[ib(idxd, p)], zdb[p], semd[p])

        def quad_body(qb, _):
            for p in range(D):
                b = D * qb + p
                zs, zd = zsb[p], zdb[p]
                pltpu.make_async_copy(
                    z2_hbm.at[ib(idxs, b)], zs, sems[p]).wait()
                pltpu.make_async_copy(
                    z2_hbm.at[ib(idxd, b)], zd, semd[p]).wait()
                res = jnp.zeros((16,), jnp.float32)
                for e in range(G):
                    acc = zs[e, pl.ds(0, 16)] * zd[e, pl.ds(0, 16)]
                    for k16 in range(1, 16):
                        acc = acc + (zs[e, pl.ds(k16 * 16, 16)] *
                                     zd[e, pl.ds(k16 * 16, 16)])
                    res = jnp.where(iota == e, jnp.sum(acc), res)
                outbuf[pl.ds(b * G, 16)] = res

                @pl.when(b + D < NBT)
                def _():
                    pltpu.async_copy(z2_hbm.at[ib(idxs, b + D)], zs, sems[p])
                    pltpu.async_copy(z2_hbm.at[ib(idxd, b + D)], zd, semd[p])
            return 0
        lax.fori_loop(0, NBT // D, quad_body, 0)

        pltpu.sync_copy(outbuf, out_hbm.at[pl.ds(w * EB, EB)])

    return dec_kernel


def _tc_a_body(x_ref, w_ref, degp_ref, hs_ref, dinv_ref):
    deg = jnp.sum(degp_ref[:, :, 0], axis=0) + 1.0
    dinv = lax.rsqrt(deg)
    h = jnp.dot(x_ref[...], w_ref[...], preferred_element_type=jnp.float32)
    hs_ref[...] = h * dinv[:, None]
    dinv_ref[...] = jnp.broadcast_to(
        jnp.pad(dinv, (0, 1024 - NBS))[None, None, :], (1, 8, 1024))


def _tc_a(x, w1, degp):
    return pl.pallas_call(
        _tc_a_body,
        grid=(NB, 4),
        in_specs=[
            pl.BlockSpec((NBS, 256), lambda nb, fc: (nb, 0)),
            pl.BlockSpec((256, 128), lambda nb, fc: (0, fc)),
            pl.BlockSpec((NC, NBS, 16), lambda nb, fc: (0, nb, 0)),
        ],
        out_specs=[
            pl.BlockSpec((NBS, 128), lambda nb, fc: (fc * NB + nb, 0)),
            pl.BlockSpec((1, 8, 1024), lambda nb, fc: (nb, 0, 0)),
        ],
        out_shape=[
            jax.ShapeDtypeStruct((4 * N, 128), jnp.float32),
            jax.ShapeDtypeStruct((NB, 8, 1024), jnp.float32),
        ],
    )(x, w1, degp)


def _tc_b_body(agg_ref, hs_ref, dinv_ref, b1_ref, w2_ref, out_ref):
    fc = pl.program_id(2)
    dinv = dinv_ref[0, 0, :NBS]
    b1row = b1_ref[fc, :]
    z = jnp.maximum(
        (agg_ref[...] + hs_ref[...]) * dinv[:, None] + b1row[None, :], 0.0)
    part = jnp.dot(z, w2_ref[...], preferred_element_type=jnp.float32)

    @pl.when(fc == 0)
    def _():
        out_ref[...] = part

    @pl.when(fc > 0)
    def _():
        out_ref[...] += part

    @pl.when(fc == 3)
    def _():
        out_ref[...] *= dinv[:, None]


def _tc_b(agg1, hs1, dinv2d, b1_2d, w2):
    return pl.pallas_call(
        _tc_b_body,
        grid=(NB, 2, 4),
        in_specs=[
            pl.BlockSpec((NBS, 128), lambda nb, oc, fc: (fc * NB + nb, 0)),
            pl.BlockSpec((NBS, 128), lambda nb, oc, fc: (fc * NB + nb, 0)),
            pl.BlockSpec((1, 8, 1024), lambda nb, oc, fc: (nb, 0, 0)),
            pl.BlockSpec((4, 128), lambda nb, oc, fc: (0, 0)),
            pl.BlockSpec((128, 128), lambda nb, oc, fc: (fc, oc)),
        ],
        out_specs=pl.BlockSpec((NBS, 128), lambda nb, oc, fc: (oc * NB + nb, 0)),
        out_shape=jax.ShapeDtypeStruct((2 * N, 128), jnp.float32),
    )(agg1, hs1, dinv2d, b1_2d, w2)


def _tc_c_body(agg_ref, hs_ref, dinv_ref, b2_ref, out_ref):
    dinv = dinv_ref[0, 0, :NBS]
    oc = pl.program_id(1)
    out_ref[...] = ((agg_ref[...] + hs_ref[...]) * dinv[:, None]
                    + b2_ref[oc, :][None, :])


def _tc_c(agg2, hs2, dinv2d, b2_2d):
    return pl.pallas_call(
        _tc_c_body,
        grid=(NB, 2),
        in_specs=[
            pl.BlockSpec((NBS, 128), lambda nb, oc: (oc * NB + nb, 0)),
            pl.BlockSpec((NBS, 128), lambda nb, oc: (oc * NB + nb, 0)),
            pl.BlockSpec((1, 8, 1024), lambda nb, oc: (nb, 0, 0)),
            pl.BlockSpec((2, 128), lambda nb, oc: (0, 0)),
        ],
        out_specs=pl.BlockSpec((NBS, 128), lambda nb, oc: (nb, oc)),
        out_shape=jax.ShapeDtypeStruct((N, 256), jnp.float32),
    )(agg2, hs2, dinv2d, b2_2d)


_deg_kernel = _sc_deg()
_agg4_kernel = _sc_agg(4)
_agg2_kernel = _sc_agg(2)
_dec_kernel = _sc_decode()


def kernel(x, edge_index, edge_label_index, W1, b1, W2, b2):
    pad = EP - E
    zpad = jnp.zeros((pad,), jnp.int32)
    src2d = jnp.concatenate([edge_index[0], zpad]).reshape(EP // 64, 64)
    dst2d = jnp.concatenate(
        [edge_index[1], jnp.full((pad,), N, jnp.int32)]).reshape(EP // 64, 64)
    els2d = jnp.concatenate([edge_label_index[0], zpad]).reshape(EROWS, 128)
    eld2d = jnp.concatenate([edge_label_index[1], zpad]).reshape(EROWS, 128)
    b1_2d = b1.reshape(4, 128)
    b2_2d = b2.reshape(2, 128)

    degp = _deg_kernel(dst2d)
    hs1, dinv2d = _tc_a(x, W1, degp)
    agg1 = _agg4_kernel(hs1, src2d, dst2d)
    hs2 = _tc_b(agg1, hs1, dinv2d, b1_2d, W2)
    agg2 = _agg2_kernel(hs2, src2d, dst2d)
    z2 = _tc_c(agg2, hs2, dinv2d, b2_2d)
    dots = _dec_kernel(z2, els2d, eld2d)
    return dots[:E]


# speedup vs baseline: 4.7113x; 1.0349x over previous
import functools

import jax
import jax.numpy as jnp
from jax import lax
from jax.experimental import pallas as pl
from jax.experimental.pallas import tpu as pltpu
from jax.experimental.pallas import tpu_sc as plsc

N = 10000
E = 160000
EROWS = 1280
EP = EROWS * 128
NACC = 10112
NB = 10
NBS = 1000
NC = 2
NS = 16

_mesh = plsc.VectorSubcoreMesh(
    core_axis_name="c", subcore_axis_name="s", num_cores=NC, num_subcores=NS)


def _sc_deg():
    RT = EROWS // (NC * NS)
    ZR = NACC // NS

    @functools.partial(
        pl.kernel,
        mesh=_mesh,
        compiler_params=pltpu.CompilerParams(needs_layout_passes=False),
        out_type=jax.ShapeDtypeStruct((NC, NACC, 16), jnp.float32),
        scratch_types=[
            pltpu.VMEM_SHARED((NACC, 16), jnp.float32),
            pltpu.VMEM((ZR, 16), jnp.float32),
            pltpu.VMEM((128, 16), jnp.float32),
            pltpu.VMEM((RT, 128), jnp.int32),
            pltpu.SemaphoreType.DMA,
        ],
    )
    def deg_kernel(dst_hbm, out_hbm, acc, zbuf, vals, idx, sem):
        c = lax.axis_index("c")
        s = lax.axis_index("s")
        w = s * NC + c

        def init_body(i, _):
            zbuf[i, :] = jnp.zeros((16,), jnp.float32)
            return 0
        lax.fori_loop(0, ZR, init_body, 0)

        def ones_body(i, _):
            vals[i, :] = jnp.ones((16,), jnp.float32)
            return 0
        lax.fori_loop(0, 128, ones_body, 0)

        pltpu.sync_copy(zbuf, acc.at[pl.ds(s * ZR, ZR)])
        plsc.subcore_barrier()

        pltpu.sync_copy(dst_hbm.at[pl.ds(w * RT, RT)], idx)

        def edge_body(j, _):
            pltpu.async_copy(vals, acc.at[idx.at[j]], sem, add=True).wait()
            return 0
        lax.fori_loop(0, RT, edge_body, 0)

        plsc.subcore_barrier()
        pltpu.sync_copy(acc.at[pl.ds(s * ZR, ZR)],
                        out_hbm.at[c].at[pl.ds(s * ZR, ZR)])

    return deg_kernel


def _sc_agg(C):
    CPC = C // NC
    RT = EROWS // NS
    K = 16
    NSB = RT // K
    ZR = 96
    ZT = NACC // NS
    WB = 624

    @functools.partial(
        pl.kernel,
        mesh=_mesh,
        compiler_params=pltpu.CompilerParams(needs_layout_passes=False),
        out_type=jax.ShapeDtypeStruct((C * N, 128), jnp.float32),
        scratch_types=[
            pltpu.VMEM_SHARED((NACC, 128), jnp.float32),
            pltpu.VMEM((ZR, 128), jnp.float32),
            pltpu.VMEM((K, 128), jnp.int32),
            pltpu.VMEM((K, 128), jnp.int32),
            pltpu.VMEM((128, 128), jnp.float32),
            pltpu.VMEM((128, 128), jnp.float32),
            pltpu.SemaphoreType.DMA,
            pltpu.SemaphoreType.DMA,
            pltpu.SemaphoreType.DMA,
            pltpu.SemaphoreType.DMA,
        ],
    )
    def agg_kernel(hs_hbm, src_hbm, dst_hbm, out_hbm,
                   acc, zbuf, idxs, idxd, rows0, rows1,
                   semg0, semg1, sems0, sems1):
        c = lax.axis_index("c")
        s = lax.axis_index("s")
        bufs = (rows0, rows1)

        def zb_body(i, _):
            for k8 in range(8):
                zbuf[i, pl.ds(k8 * 16, 16)] = jnp.zeros((16,), jnp.float32)
            return 0
        lax.fori_loop(0, ZR, zb_body, 0)

        for chunk in range(CPC):
            fc = c * CPC + chunk
            for z in range(6):
                pltpu.sync_copy(zbuf, acc.at[pl.ds(s * ZT + z * ZR, ZR)])
            pltpu.sync_copy(zbuf.at[pl.ds(0, 56)],
                            acc.at[pl.ds(s * ZT + 6 * ZR, 56)])
            plsc.subcore_barrier()

            off = fc * N

            def sb_body(t, _):
                r0 = s * RT + t * K
                pltpu.sync_copy(src_hbm.at[pl.ds(r0, K)], idxs)
                pltpu.sync_copy(dst_hbm.at[pl.ds(r0, K)], idxd)
                for j in range(K):
                    for k8 in range(8):
                        idxs[j, pl.ds(k8 * 16, 16)] = (
                            idxs[j, pl.ds(k8 * 16, 16)] + off)
                g = [None] * K
                g[0] = pltpu.async_copy(hs_hbm.at[idxs.at[0]], rows0, semg0)
                g[1] = pltpu.async_copy(hs_hbm.at[idxs.at[1]], rows1, semg1)
                for jp in range(K // 2):
                    j0, j1 = 2 * jp, 2 * jp + 1
                    g[j0].wait()
                    s0 = pltpu.async_copy(
                        rows0, acc.at[idxd.at[j0]], sems0, add=True)
                    g[j1].wait()
                    s1 = pltpu.async_copy(
                        rows1, acc.at[idxd.at[j1]], sems1, add=True)
                    s0.wait()
                    if j0 + 2 < K:
                        g[j0 + 2] = pltpu.async_copy(
                            hs_hbm.at[idxs.at[j0 + 2]], rows0, semg0)
                    s1.wait()
                    if j1 + 2 < K:
                        g[j1 + 2] = pltpu.async_copy(
                            hs_hbm.at[idxs.at[j1 + 2]], rows1, semg1)
                return 0
            lax.fori_loop(0, NSB, sb_body, 0)

            plsc.subcore_barrier()
            pltpu.sync_copy(acc.at[pl.ds(s * WB, WB)],
                            out_hbm.at[pl.ds(off + s * WB, WB)])

            @pl.when(s == NS - 1)
            def _():
                pltpu.sync_copy(acc.at[pl.ds(NS * WB, N - NS * WB)],
                                out_hbm.at[pl.ds(off + NS * WB, N - NS * WB)])
            if chunk + 1 < CPC:
                plsc.subcore_barrier()

    return agg_kernel


def _sc_decode():
    G = 32
    RT = EP // (NC * NS * G)
    EB = RT * G

    @functools.partial(
        pl.kernel,
        mesh=_mesh,
        compiler_params=pltpu.CompilerParams(needs_layout_passes=False),
        out_type=jax.ShapeDtypeStruct((EP,), jnp.float32),
        scratch_types=[
            pltpu.VMEM((RT, G), jnp.int32),
            pltpu.VMEM((RT, G), jnp.int32),
            pltpu.VMEM((G, 256), jnp.float32),
            pltpu.VMEM((G, 256), jnp.float32),
            pltpu.VMEM((G, 256), jnp.float32),
            pltpu.VMEM((G, 256), jnp.float32),
            pltpu.VMEM((EB,), jnp.float32),
            pltpu.SemaphoreType.DMA,
            pltpu.SemaphoreType.DMA,
            pltpu.SemaphoreType.DMA,
            pltpu.SemaphoreType.DMA,
        ],
    )
    def dec_kernel(z2_hbm, els_hbm, eld_hbm, out_hbm,
                   idxs, idxd, zs0, zd0, zs1, zd1, outbuf,
                   sems0, sems1, semd0, semd1):
        c = lax.axis_index("c")
        s = lax.axis_index("s")
        w = s * NC + c
        pltpu.sync_copy(els_hbm.at[pl.ds(w * RT, RT)], idxs)
        pltpu.sync_copy(eld_hbm.at[pl.ds(w * RT, RT)], idxd)

        iota = lax.iota(jnp.int32, 16)

        pltpu.async_copy(z2_hbm.at[idxs.at[0]], zs0, sems0)
        pltpu.async_copy(z2_hbm.at[idxd.at[0]], zd0, semd0)
        pltpu.async_copy(z2_hbm.at[idxs.at[1]], zs1, sems1)
        pltpu.async_copy(z2_hbm.at[idxd.at[1]], zd1, semd1)

        def pair_body(bp, _):
            for phase in range(2):
                b = 2 * bp + phase
                zs = zs0 if phase == 0 else zs1
                zd = zd0 if phase == 0 else zd1
                sems = sems0 if phase == 0 else sems1
                semd = semd0 if phase == 0 else semd1
                pltpu.make_async_copy(z2_hbm.at[idxs.at[b]], zs, sems).wait()
                pltpu.make_async_copy(z2_hbm.at[idxd.at[b]], zd, semd).wait()
                for g in range(G // 16):
                    res = jnp.zeros((16,), jnp.float32)
                    for e in range(16):
                        er = g * 16 + e
                        acc = zs[er, pl.ds(0, 16)] * zd[er, pl.ds(0, 16)]
                        for k16 in range(1, 16):
                            acc = acc + (zs[er, pl.ds(k16 * 16, 16)] *
                                         zd[er, pl.ds(k16 * 16, 16)])
                        res = jnp.where(iota == e, jnp.sum(acc), res)
                    outbuf[pl.ds(b * G + g * 16, 16)] = res

                @pl.when(b + 2 < RT)
                def _():
                    pltpu.async_copy(z2_hbm.at[idxs.at[b + 2]], zs, sems)
                    pltpu.async_copy(z2_hbm.at[idxd.at[b + 2]], zd, semd)
            return 0
        lax.fori_loop(0, RT // 2, pair_body, 0)

        pltpu.sync_copy(outbuf, out_hbm.at[pl.ds(w * EB, EB)])

    return dec_kernel


def _tc_a_body(x_ref, w_ref, degp_ref, hs_ref, dinv_ref):
    deg = jnp.sum(degp_ref[:, :, 0], axis=0) + 1.0
    dinv = lax.rsqrt(deg)
    h = jnp.dot(x_ref[...], w_ref[...], preferred_element_type=jnp.float32)
    hs_ref[...] = h * dinv[:, None]
    dinv_ref[...] = jnp.broadcast_to(
        jnp.pad(dinv, (0, 1024 - NBS))[None, None, :], (1, 8, 1024))


def _tc_a(x, w1, degp):
    return pl.pallas_call(
        _tc_a_body,
        grid=(NB, 4),
        in_specs=[
            pl.BlockSpec((NBS, 256), lambda nb, fc: (nb, 0)),
            pl.BlockSpec((256, 128), lambda nb, fc: (0, fc)),
            pl.BlockSpec((NC, NBS, 16), lambda nb, fc: (0, nb, 0)),
        ],
        out_specs=[
            pl.BlockSpec((NBS, 128), lambda nb, fc: (fc * NB + nb, 0)),
            pl.BlockSpec((1, 8, 1024), lambda nb, fc: (nb, 0, 0)),
        ],
        out_shape=[
            jax.ShapeDtypeStruct((4 * N, 128), jnp.float32),
            jax.ShapeDtypeStruct((NB, 8, 1024), jnp.float32),
        ],
    )(x, w1, degp)


def _tc_b_body(agg_ref, hs_ref, dinv_ref, b1_ref, w2_ref, out_ref):
    fc = pl.program_id(2)
    dinv = dinv_ref[0, 0, :NBS]
    b1row = b1_ref[fc, :]
    z = jnp.maximum(
        (agg_ref[...] + hs_ref[...]) * dinv[:, None] + b1row[None, :], 0.0)
    part = jnp.dot(z, w2_ref[...], preferred_element_type=jnp.float32)

    @pl.when(fc == 0)
    def _():
        out_ref[...] = part

    @pl.when(fc > 0)
    def _():
        out_ref[...] += part

    @pl.when(fc == 3)
    def _():
        out_ref[...] *= dinv[:, None]


def _tc_b(agg1, hs1, dinv2d, b1_2d, w2):
    return pl.pallas_call(
        _tc_b_body,
        grid=(NB, 2, 4),
        in_specs=[
            pl.BlockSpec((NBS, 128), lambda nb, oc, fc: (fc * NB + nb, 0)),
            pl.BlockSpec((NBS, 128), lambda nb, oc, fc: (fc * NB + nb, 0)),
            pl.BlockSpec((1, 8, 1024), lambda nb, oc, fc: (nb, 0, 0)),
            pl.BlockSpec((4, 128), lambda nb, oc, fc: (0, 0)),
            pl.BlockSpec((128, 128), lambda nb, oc, fc: (fc, oc)),
        ],
        out_specs=pl.BlockSpec((NBS, 128), lambda nb, oc, fc: (oc * NB + nb, 0)),
        out_shape=jax.ShapeDtypeStruct((2 * N, 128), jnp.float32),
    )(agg1, hs1, dinv2d, b1_2d, w2)


def _tc_c_body(agg_ref, hs_ref, dinv_ref, b2_ref, out_ref):
    dinv = dinv_ref[0, 0, :NBS]
    oc = pl.program_id(1)
    out_ref[...] = ((agg_ref[...] + hs_ref[...]) * dinv[:, None]
                    + b2_ref[oc, :][None, :])


def _tc_c(agg2, hs2, dinv2d, b2_2d):
    return pl.pallas_call(
        _tc_c_body,
        grid=(NB, 2),
        in_specs=[
            pl.BlockSpec((NBS, 128), lambda nb, oc: (oc * NB + nb, 0)),
            pl.BlockSpec((NBS, 128), lambda nb, oc: (oc * NB + nb, 0)),
            pl.BlockSpec((1, 8, 1024), lambda nb, oc: (nb, 0, 0)),
            pl.BlockSpec((2, 128), lambda nb, oc: (0, 0)),
        ],
        out_specs=pl.BlockSpec((NBS, 128), lambda nb, oc: (nb, oc)),
        out_shape=jax.ShapeDtypeStruct((N, 256), jnp.float32),
    )(agg2, hs2, dinv2d, b2_2d)


_deg_kernel = _sc_deg()
_agg4_kernel = _sc_agg(4)
_agg2_kernel = _sc_agg(2)
_dec_kernel = _sc_decode()


def kernel(x, edge_index, edge_label_index, W1, b1, W2, b2):
    pad = EP - E
    zpad = jnp.zeros((pad,), jnp.int32)
    src2d = jnp.concatenate([edge_index[0], zpad]).reshape(EROWS, 128)
    dst2d = jnp.concatenate(
        [edge_index[1], jnp.full((pad,), N, jnp.int32)]).reshape(EROWS, 128)
    els2d = jnp.concatenate([edge_label_index[0], zpad]).reshape(EP // 32, 32)
    eld2d = jnp.concatenate([edge_label_index[1], zpad]).reshape(EP // 32, 32)
    b1_2d = b1.reshape(4, 128)
    b2_2d = b2.reshape(2, 128)

    degp = _deg_kernel(dst2d)
    hs1, dinv2d = _tc_a(x, W1, degp)
    agg1 = _agg4_kernel(hs1, src2d, dst2d)
    hs2 = _tc_b(agg1, hs1, dinv2d, b1_2d, W2)
    agg2 = _agg2_kernel(hs2, src2d, dst2d)
    z2 = _tc_c(agg2, hs2, dinv2d, b2_2d)
    dots = _dec_kernel(z2, els2d, eld2d)
    return dots[:E]
